# Initial kernel scaffold; baseline (speedup 1.0000x reference)
import jax
import jax.numpy as jnp
from jax.experimental import pallas as pl


def kernel(node_feats, edge_feats, i, j, W_sg, b_sg, W_dg, b_dg, W_eg, b_eg, W_su, b_su, W_du, b_du):
    raise NotImplementedError("write your pallas kernel here")


# speedup vs baseline: 1.2741x; 1.2741x over previous
import functools

import jax
import jax.numpy as jnp
from jax import lax
from jax.experimental import pallas as pl
from jax.experimental.pallas import tpu as pltpu
from jax.experimental.pallas import tpu_sc as plsc

F32 = jnp.float32
NC, NS = 2, 16
CE = 80
NODE_SUB = 25


def _node_proj_body(x, wsg, bsg, wdg, bdg, wdu, bdu, wsu, bsu,
                    osg, odg, odu, osu):
    xv = x[...]

    def proj(w, b):
        return lax.dot_general(xv, w[...], (((1,), (1,)), ((), ())),
                               preferred_element_type=F32) + b[...]

    osg[...] = proj(wsg, bsg)
    odg[...] = proj(wdg, bdg)
    odu[...] = proj(wdu, bdu)
    osu[...] = proj(wsu, bsu)


def _edge_proj_body(x, w, b, o):
    o[...] = lax.dot_general(x[...], w[...], (((1,), (1,)), ((), ())),
                             preferred_element_type=F32) + b[...]


def _sc_body(N, E, ti, tj, psut, node_t, peg, edge, i_idx, j_idx,
             yout, xout,
             acc_sig, acc_m,
             idx_i, idx_j, idx_is, idx_js,
             g_sg, g_dj, sig_b, m_b, yout_b,
             ep_sig, ep_m, ep_psu, ep_node, ep_x,
             sem0, sem1, sem2, sem3):
    c = lax.axis_index("c")
    s = lax.axis_index("s")
    cN = c * N
    c64 = c * 64
    zero16 = jnp.zeros((16,), F32)

    npt = N // NS
    ept = E // NS
    nchunks = ept // CE
    nsub = npt // NODE_SUB

    def zrow(r, _):
        for q in range(4):
            ep_sig[r, pl.ds(q * 16, 16)] = zero16
        return _
    lax.fori_loop(0, NODE_SUB, zrow, None)
    for q in range(nsub):
        pltpu.sync_copy(ep_sig, acc_sig.at[pl.ds(s * npt + q * NODE_SUB, NODE_SUB)])
        pltpu.sync_copy(ep_sig, acc_m.at[pl.ds(s * npt + q * NODE_SUB, NODE_SUB)])
    plsc.subcore_barrier()

    ebase0 = s * ept

    def chunk(k, _):
        base = ebase0 + k * CE
        pltpu.sync_copy(i_idx.at[pl.ds(base, CE)], idx_i)
        pltpu.sync_copy(j_idx.at[pl.ds(base, CE)], idx_j)

        def shift(r, x_):
            sl = pl.ds(r * 16, 16)
            idx_is[sl] = idx_i[sl] + cN
            idx_js[sl] = idx_j[sl] + cN
            return x_
        lax.fori_loop(0, CE // 16, shift, None)

        cp1 = pltpu.async_copy(ti.at[idx_is], g_sg, sem0)
        cp2 = pltpu.async_copy(tj.at[idx_js], g_dj, sem1)
        cp3 = pltpu.async_copy(peg.at[pl.ds(base, CE), pl.ds(c64, 64)], sig_b, sem2)
        cp4 = pltpu.async_copy(edge.at[pl.ds(base, CE), pl.ds(c64, 64)], yout_b, sem3)
        cp1.wait()
        cp2.wait()
        cp3.wait()
        cp4.wait()

        def row(r, x_):
            for q in range(4):
                sl = pl.ds(q * 16, 16)
                ysum = g_sg[r, sl] + g_dj[r, pl.ds(q * 16, 16)] + sig_b[r, sl]
                sg = 1.0 / (1.0 + jnp.exp(-ysum))
                sig_b[r, sl] = sg
                m_b[r, sl] = g_dj[r, pl.ds(64 + q * 16, 16)] * sg
                yout_b[r, sl] = yout_b[r, sl] + ysum * sg
            return x_
        lax.fori_loop(0, CE, row, None)

        pltpu.sync_copy(yout_b, yout.at[pl.ds(base, CE), pl.ds(c64, 64)])
        pltpu.sync_copy(sig_b, acc_sig.at[idx_i], add=True)
        pltpu.sync_copy(m_b, acc_m.at[idx_i], add=True)
        return _

    lax.fori_loop(0, nchunks, chunk, None)
    plsc.subcore_barrier()

    nbase0 = s * npt

    def ep(q, _):
        nb = nbase0 + q * NODE_SUB
        pltpu.sync_copy(acc_sig.at[pl.ds(nb, NODE_SUB)], ep_sig)
        pltpu.sync_copy(acc_m.at[pl.ds(nb, NODE_SUB)], ep_m)
        pltpu.sync_copy(psut.at[pl.ds(cN + nb, NODE_SUB)], ep_psu)
        pltpu.sync_copy(node_t.at[pl.ds(cN + nb, NODE_SUB)], ep_node)

        def nrow(r, x_):
            for q4 in range(4):
                sl = pl.ds(q4 * 16, 16)
                h = ep_m[r, sl] / (ep_sig[r, sl] + 1e-6)
                t = ep_psu[r, sl] + h
                st = 1.0 / (1.0 + jnp.exp(-t))
                ep_x[r, sl] = ep_node[r, sl] + t * st
            return x_
        lax.fori_loop(0, NODE_SUB, nrow, None)

        pltpu.sync_copy(ep_x, xout.at[pl.ds(cN + nb, NODE_SUB)])
        return _

    lax.fori_loop(0, nsub, ep, None)


def kernel(node_feats, edge_feats, i, j, W_sg, b_sg, W_dg, b_dg,
           W_eg, b_eg, W_su, b_su, W_du, b_du):
    B, N, D = node_feats.shape
    E = edge_feats.shape[1]
    node2 = node_feats.reshape(N, D)
    edge2 = edge_feats.reshape(E, D)
    i32 = i.astype(jnp.int32)
    j32 = j.astype(jnp.int32)
    h = D // 2

    BN = 400
    wspec = pl.BlockSpec((D, D), lambda nb: (0, 0))
    bspec = pl.BlockSpec((1, D), lambda nb: (0, 0))
    outs = pl.pallas_call(
        _node_proj_body,
        grid=(N // BN,),
        in_specs=[pl.BlockSpec((BN, D), lambda nb: (nb, 0)),
                  wspec, bspec, wspec, bspec, wspec, bspec, wspec, bspec],
        out_specs=[pl.BlockSpec((BN, D), lambda nb: (nb, 0))] * 4,
        out_shape=[jax.ShapeDtypeStruct((N, D), F32)] * 4,
    )(node2, W_sg, b_sg.reshape(1, D), W_dg, b_dg.reshape(1, D),
      W_du, b_du.reshape(1, D), W_su, b_su.reshape(1, D))
    p_sg, p_dg, p_du, p_su = outs

    ti = p_sg.reshape(N, 2, h).transpose(1, 0, 2).reshape(2 * N, h)
    tj = jnp.concatenate([p_dg.reshape(N, 2, h), p_du.reshape(N, 2, h)],
                         axis=2).transpose(1, 0, 2).reshape(2 * N, D)
    psut = p_su.reshape(N, 2, h).transpose(1, 0, 2).reshape(2 * N, h)
    node_t = node2.reshape(N, 2, h).transpose(1, 0, 2).reshape(2 * N, h)

    BE = 1000
    peg = pl.pallas_call(
        _edge_proj_body,
        grid=(E // BE,),
        in_specs=[pl.BlockSpec((BE, D), lambda nb: (nb, 0)),
                  pl.BlockSpec((D, D), lambda nb: (0, 0)),
                  pl.BlockSpec((1, D), lambda nb: (0, 0))],
        out_specs=pl.BlockSpec((BE, D), lambda nb: (nb, 0)),
        out_shape=jax.ShapeDtypeStruct((E, D), F32),
    )(edge2, W_eg, b_eg.reshape(1, D))

    mesh = plsc.VectorSubcoreMesh(core_axis_name="c", subcore_axis_name="s",
                                  num_cores=NC, num_subcores=NS)
    sc_fn = pl.kernel(
        functools.partial(_sc_body, N, E),
        out_type=[jax.ShapeDtypeStruct((E, D), F32),
                  jax.ShapeDtypeStruct((2 * N, h), F32)],
        mesh=mesh,
        compiler_params=pltpu.CompilerParams(use_tc_tiling_on_sc=False),
        scratch_types=[
            pltpu.VMEM_SHARED((N, h), F32),
            pltpu.VMEM_SHARED((N, h), F32),
            pltpu.VMEM((CE,), jnp.int32),
            pltpu.VMEM((CE,), jnp.int32),
            pltpu.VMEM((CE,), jnp.int32),
            pltpu.VMEM((CE,), jnp.int32),
            pltpu.VMEM((CE, h), F32),
            pltpu.VMEM((CE, D), F32),
            pltpu.VMEM((CE, h), F32),
            pltpu.VMEM((CE, h), F32),
            pltpu.VMEM((CE, h), F32),
            pltpu.VMEM((NODE_SUB, h), F32),
            pltpu.VMEM((NODE_SUB, h), F32),
            pltpu.VMEM((NODE_SUB, h), F32),
            pltpu.VMEM((NODE_SUB, h), F32),
            pltpu.VMEM((NODE_SUB, h), F32),
            pltpu.SemaphoreType.DMA,
            pltpu.SemaphoreType.DMA,
            pltpu.SemaphoreType.DMA,
            pltpu.SemaphoreType.DMA,
        ],
    )
    yout, xout = sc_fn(ti, tj, psut, node_t, peg, edge2, i32, j32)

    x = xout.reshape(2, N, h).transpose(1, 0, 2).reshape(B, N, D)
    y = yout.reshape(B, E, D)
    return (x, y)


# speedup vs baseline: 1.2981x; 1.0188x over previous
import functools

import jax
import jax.numpy as jnp
from jax import lax
from jax.experimental import pallas as pl
from jax.experimental.pallas import tpu as pltpu
from jax.experimental.pallas import tpu_sc as plsc

F32 = jnp.float32
NC, NS = 2, 16
CE = 40
NODE_SUB = 25


def _node_proj_body(x, wsg, bsg, wdg, bdg, wdu, bdu, wsu, bsu,
                    osg, odg, odu, osu):
    xv = x[...]

    def proj(w, b):
        return lax.dot_general(xv, w[...], (((1,), (1,)), ((), ())),
                               preferred_element_type=F32) + b[...]

    osg[...] = proj(wsg, bsg)
    odg[...] = proj(wdg, bdg)
    odu[...] = proj(wdu, bdu)
    osu[...] = proj(wsu, bsu)


def _edge_proj_body(x, w, b, o):
    o[...] = lax.dot_general(x[...], w[...], (((1,), (1,)), ((), ())),
                             preferred_element_type=F32) + b[...]


def _sc_body(N, E, ti0, ti1, tj0, tj1, psut, node_t, peg, edge, i_hbm, j_hbm,
             yout, xout,
             acc_sig, acc_m,
             ib0, ib1, ib2, ib3, jb0, jb1, jb2, jb3,
             g_sg0, g_sg1, g_dj0, g_dj1,
             sig0, sig1, m0, m1, yo0, yo1,
             ep_sig, ep_m, ep_psu, ep_node,
             sem_ij, sem_in0, sem_in1, sem_out0, sem_out1):
    c = lax.axis_index("c")
    s = lax.axis_index("s")
    cN = c * N
    c64 = c * 64
    zero16 = jnp.zeros((16,), F32)

    ib = (ib0, ib1, ib2, ib3)
    jb = (jb0, jb1, jb2, jb3)
    g_sg = (g_sg0, g_sg1)
    g_dj = (g_dj0, g_dj1)
    sig = (sig0, sig1)
    mb = (m0, m1)
    yo = (yo0, yo1)
    sem_in = (sem_in0, sem_in1)
    sem_out = (sem_out0, sem_out1)

    npt = N // NS
    ept = E // NS
    nchunks = ept // CE
    nsub = npt // NODE_SUB
    ebase0 = s * ept

    def zrow(r, x_):
        for q in range(4):
            ep_sig[r, pl.ds(q * 16, 16)] = zero16
        return x_
    lax.fori_loop(0, NODE_SUB, zrow, None)
    for q in range(nsub):
        pltpu.sync_copy(ep_sig, acc_sig.at[pl.ds(s * npt + q * NODE_SUB, NODE_SUB)])
        pltpu.sync_copy(ep_sig, acc_m.at[pl.ds(s * npt + q * NODE_SUB, NODE_SUB)])
    plsc.subcore_barrier()

    def issue_ij(k, q4):
        base = ebase0 + k * CE
        pltpu.async_copy(i_hbm.at[pl.ds(base, CE)], ib[q4], sem_ij)
        pltpu.async_copy(j_hbm.at[pl.ds(base, CE)], jb[q4], sem_ij)

    def wait_ij(q4):
        pltpu.make_async_copy(i_hbm.at[pl.ds(0, CE)], ib[q4], sem_ij).wait()
        pltpu.make_async_copy(j_hbm.at[pl.ds(0, CE)], jb[q4], sem_ij).wait()

    def issue_in(k, p, q4):
        base = ebase0 + k * CE

        @pl.when(c == 0)
        def _():
            pltpu.async_copy(ti0.at[ib[q4]], g_sg[p], sem_in[p])
            pltpu.async_copy(tj0.at[jb[q4]], g_dj[p], sem_in[p])

        @pl.when(c == 1)
        def _():
            pltpu.async_copy(ti1.at[ib[q4]], g_sg[p], sem_in[p])
            pltpu.async_copy(tj1.at[jb[q4]], g_dj[p], sem_in[p])

        pltpu.async_copy(peg.at[pl.ds(base, CE), pl.ds(c64, 64)], sig[p], sem_in[p])
        pltpu.async_copy(edge.at[pl.ds(base, CE), pl.ds(c64, 64)], yo[p], sem_in[p])

    def wait_in(p):
        pltpu.make_async_copy(ti0.at[ib[0]], g_sg[p], sem_in[p]).wait()
        pltpu.make_async_copy(tj0.at[jb[0]], g_dj[p], sem_in[p]).wait()
        pltpu.make_async_copy(peg.at[pl.ds(0, CE), pl.ds(c64, 64)], sig[p], sem_in[p]).wait()
        pltpu.make_async_copy(edge.at[pl.ds(0, CE), pl.ds(c64, 64)], yo[p], sem_in[p]).wait()

    def compute(p):
        def row(r, x_):
            for q in range(4):
                sl = pl.ds(q * 16, 16)
                ysum = g_sg[p][r, sl] + g_dj[p][r, pl.ds(q * 16, 16)] + sig[p][r, sl]
                sg = 1.0 / (1.0 + jnp.exp(-ysum))
                sig[p][r, sl] = sg
                mb[p][r, sl] = g_dj[p][r, pl.ds(64 + q * 16, 16)] * sg
                yo[p][r, sl] = yo[p][r, sl] + ysum * sg
            return x_
        lax.fori_loop(0, CE, row, None)

    def issue_out(k, p, q4):
        base = ebase0 + k * CE
        pltpu.async_copy(yo[p], yout.at[pl.ds(base, CE), pl.ds(c64, 64)], sem_out[p])
        pltpu.sync_copy(sig[p], acc_sig.at[ib[q4]], add=True)
        pltpu.sync_copy(mb[p], acc_m.at[ib[q4]], add=True)

    def wait_out(p):
        pltpu.make_async_copy(yo[p], yout.at[pl.ds(0, CE), pl.ds(c64, 64)], sem_out[p]).wait()

    issue_ij(0, 0)
    issue_ij(1, 1)
    wait_ij(0)
    issue_in(0, 0, 0)

    wait_in(0)
    compute(0)
    issue_out(0, 0, 0)
    wait_ij(1)
    issue_in(1, 1, 1)
    issue_ij(2, 2)

    wait_in(1)
    compute(1)
    issue_out(1, 1, 1)
    wait_ij(2)
    wait_out(0)
    issue_in(2, 0, 2)
    issue_ij(3, 3)

    def superstep(t, x_):
        k0 = 2 + t * 4

        def stage(koff, p, q4):
            k = k0 + koff
            wait_in(p)
            compute(p)
            issue_out(k, p, q4)
            wait_ij((q4 + 1) % 4)
            wait_out(1 - p)
            issue_in(k + 1, 1 - p, (q4 + 1) % 4)
            issue_ij(k + 2, (q4 + 2) % 4)

        stage(0, 0, 2)
        stage(1, 1, 3)
        stage(2, 0, 0)
        stage(3, 1, 1)
        return x_

    lax.fori_loop(0, (nchunks - 4) // 4, superstep, None)

    wait_in(0)
    compute(0)
    issue_out(nchunks - 2, 0, 2)
    wait_ij(3)
    wait_out(1)
    issue_in(nchunks - 1, 1, 3)

    wait_in(1)
    compute(1)
    issue_out(nchunks - 1, 1, 3)

    wait_out(0)
    wait_out(1)
    plsc.subcore_barrier()

    nbase0 = s * npt

    def ep(q, x_):
        nb = nbase0 + q * NODE_SUB
        pltpu.sync_copy(acc_sig.at[pl.ds(nb, NODE_SUB)], ep_sig)
        pltpu.sync_copy(acc_m.at[pl.ds(nb, NODE_SUB)], ep_m)
        pltpu.sync_copy(psut.at[pl.ds(cN + nb, NODE_SUB)], ep_psu)
        pltpu.sync_copy(node_t.at[pl.ds(cN + nb, NODE_SUB)], ep_node)

        def nrow(r, y_):
            for q4 in range(4):
                sl = pl.ds(q4 * 16, 16)
                hh = ep_m[r, sl] / (ep_sig[r, sl] + 1e-6)
                t = ep_psu[r, sl] + hh
                st = 1.0 / (1.0 + jnp.exp(-t))
                ep_node[r, sl] = ep_node[r, sl] + t * st
            return y_
        lax.fori_loop(0, NODE_SUB, nrow, None)

        pltpu.sync_copy(ep_node, xout.at[pl.ds(cN + nb, NODE_SUB)])
        return x_

    lax.fori_loop(0, nsub, ep, None)


def kernel(node_feats, edge_feats, i, j, W_sg, b_sg, W_dg, b_dg,
           W_eg, b_eg, W_su, b_su, W_du, b_du):
    B, N, D = node_feats.shape
    E = edge_feats.shape[1]
    node2 = node_feats.reshape(N, D)
    edge2 = edge_feats.reshape(E, D)
    i32 = i.astype(jnp.int32)
    j32 = j.astype(jnp.int32)
    h = D // 2

    BN = 400
    wspec = pl.BlockSpec((D, D), lambda nb: (0, 0))
    bspec = pl.BlockSpec((1, D), lambda nb: (0, 0))
    outs = pl.pallas_call(
        _node_proj_body,
        grid=(N // BN,),
        in_specs=[pl.BlockSpec((BN, D), lambda nb: (nb, 0)),
                  wspec, bspec, wspec, bspec, wspec, bspec, wspec, bspec],
        out_specs=[pl.BlockSpec((BN, D), lambda nb: (nb, 0))] * 4,
        out_shape=[jax.ShapeDtypeStruct((N, D), F32)] * 4,
    )(node2, W_sg, b_sg.reshape(1, D), W_dg, b_dg.reshape(1, D),
      W_du, b_du.reshape(1, D), W_su, b_su.reshape(1, D))
    p_sg, p_dg, p_du, p_su = outs

    ti0 = p_sg[:, :h]
    ti1 = p_sg[:, h:]
    tj0 = jnp.concatenate([p_dg[:, :h], p_du[:, :h]], axis=1)
    tj1 = jnp.concatenate([p_dg[:, h:], p_du[:, h:]], axis=1)
    psut = p_su.reshape(N, 2, h).transpose(1, 0, 2).reshape(2 * N, h)
    node_t = node2.reshape(N, 2, h).transpose(1, 0, 2).reshape(2 * N, h)

    BE = 1000
    peg = pl.pallas_call(
        _edge_proj_body,
        grid=(E // BE,),
        in_specs=[pl.BlockSpec((BE, D), lambda nb: (nb, 0)),
                  pl.BlockSpec((D, D), lambda nb: (0, 0)),
                  pl.BlockSpec((1, D), lambda nb: (0, 0))],
        out_specs=pl.BlockSpec((BE, D), lambda nb: (nb, 0)),
        out_shape=jax.ShapeDtypeStruct((E, D), F32),
    )(edge2, W_eg, b_eg.reshape(1, D))

    mesh = plsc.VectorSubcoreMesh(core_axis_name="c", subcore_axis_name="s",
                                  num_cores=NC, num_subcores=NS)
    sc_fn = pl.kernel(
        functools.partial(_sc_body, N, E),
        out_type=[jax.ShapeDtypeStruct((E, D), F32),
                  jax.ShapeDtypeStruct((2 * N, h), F32)],
        mesh=mesh,
        compiler_params=pltpu.CompilerParams(use_tc_tiling_on_sc=False),
        scratch_types=[
            pltpu.VMEM_SHARED((N, h), F32),
            pltpu.VMEM_SHARED((N, h), F32),
            pltpu.VMEM((CE,), jnp.int32),
            pltpu.VMEM((CE,), jnp.int32),
            pltpu.VMEM((CE,), jnp.int32),
            pltpu.VMEM((CE,), jnp.int32),
            pltpu.VMEM((CE,), jnp.int32),
            pltpu.VMEM((CE,), jnp.int32),
            pltpu.VMEM((CE,), jnp.int32),
            pltpu.VMEM((CE,), jnp.int32),
            pltpu.VMEM((CE, h), F32),
            pltpu.VMEM((CE, h), F32),
            pltpu.VMEM((CE, D), F32),
            pltpu.VMEM((CE, D), F32),
            pltpu.VMEM((CE, h), F32),
            pltpu.VMEM((CE, h), F32),
            pltpu.VMEM((CE, h), F32),
            pltpu.VMEM((CE, h), F32),
            pltpu.VMEM((CE, h), F32),
            pltpu.VMEM((CE, h), F32),
            pltpu.VMEM((NODE_SUB, h), F32),
            pltpu.VMEM((NODE_SUB, h), F32),
            pltpu.VMEM((NODE_SUB, h), F32),
            pltpu.VMEM((NODE_SUB, h), F32),
            pltpu.SemaphoreType.DMA,
            pltpu.SemaphoreType.DMA,
            pltpu.SemaphoreType.DMA,
            pltpu.SemaphoreType.DMA,
            pltpu.SemaphoreType.DMA,
        ],
    )
    yout, xout = sc_fn(ti0, ti1, tj0, tj1, psut, node_t, peg, edge2, i32, j32)

    x = xout.reshape(2, N, h).transpose(1, 0, 2).reshape(B, N, D)
    y = yout.reshape(B, E, D)
    return (x, y)


# speedup vs baseline: 2.6438x; 2.0367x over previous
import functools

import jax
import jax.numpy as jnp
from jax import lax
from jax.experimental import pallas as pl
from jax.experimental.pallas import tpu as pltpu
from jax.experimental.pallas import tpu_sc as plsc

F32 = jnp.float32
NC, NS = 2, 16
CE = 40
NODE_SUB = 25


def _node_proj_body(x, wsg, bsg, wdg, bdg, wdu, bdu, wsu, bsu,
                    osg, odg, odu, osu):
    xv = x[...]

    def proj(w, b):
        return lax.dot_general(xv, w[...], (((1,), (1,)), ((), ())),
                               preferred_element_type=F32) + b[...]

    osg[...] = proj(wsg, bsg)
    odg[...] = proj(wdg, bdg)
    odu[...] = proj(wdu, bdu)
    osu[...] = proj(wsu, bsu)


def _edge_proj_body(x, w, b, o):
    o[...] = lax.dot_general(x[...], w[...], (((1,), (1,)), ((), ())),
                             preferred_element_type=F32) + b[...]


def _sc_body(N, E, ti0, ti1, tj0, tj1, psut, node_t, peg, edge, i_hbm, j_hbm,
             yout, xout,
             acc_sig, acc_m,
             ib0, ib1, ib2, ib3, jb0, jb1, jb2, jb3,
             g_sg0, g_sg1, g_dj0, g_dj1,
             sig0, sig1, m0, m1, yo0, yo1,
             ep_sig, ep_m, ep_psu, ep_node,
             sem_ij, sem_in0, sem_in1, sem_out0, sem_out1):
    c = lax.axis_index("c")
    s = lax.axis_index("s")
    cN = c * N
    c64 = c * 64
    zero16 = jnp.zeros((16,), F32)

    ib = (ib0, ib1, ib2, ib3)
    jb = (jb0, jb1, jb2, jb3)
    g_sg = (g_sg0, g_sg1)
    g_dj = (g_dj0, g_dj1)
    sig = (sig0, sig1)
    mb = (m0, m1)
    yo = (yo0, yo1)
    sem_in = (sem_in0, sem_in1)
    sem_out = (sem_out0, sem_out1)

    npt = N // NS
    ept = E // NS
    nchunks = ept // CE
    nsub = npt // NODE_SUB
    ebase0 = s * ept

    def zrow(r, x_):
        for q in range(4):
            ep_sig[r, pl.ds(q * 16, 16)] = zero16
        return x_
    lax.fori_loop(0, NODE_SUB, zrow, None)
    for q in range(nsub):
        pltpu.sync_copy(ep_sig, acc_sig.at[pl.ds(s * npt + q * NODE_SUB, NODE_SUB)])
        pltpu.sync_copy(ep_sig, acc_m.at[pl.ds(s * npt + q * NODE_SUB, NODE_SUB)])
    plsc.subcore_barrier()

    def issue_ij(k, q4):
        base = ebase0 + k * CE
        pltpu.async_copy(i_hbm.at[pl.ds(base, CE)], ib[q4], sem_ij)
        pltpu.async_copy(j_hbm.at[pl.ds(base, CE)], jb[q4], sem_ij)

    def wait_ij(q4):
        pltpu.make_async_copy(i_hbm.at[pl.ds(0, CE)], ib[q4], sem_ij).wait()
        pltpu.make_async_copy(j_hbm.at[pl.ds(0, CE)], jb[q4], sem_ij).wait()

    def issue_in(k, p, q4):
        base = ebase0 + k * CE

        @pl.when(c == 0)
        def _():
            pltpu.async_copy(ti0.at[ib[q4]], g_sg[p], sem_in[p])
            pltpu.async_copy(tj0.at[jb[q4]], g_dj[p], sem_in[p])

        @pl.when(c == 1)
        def _():
            pltpu.async_copy(ti1.at[ib[q4]], g_sg[p], sem_in[p])
            pltpu.async_copy(tj1.at[jb[q4]], g_dj[p], sem_in[p])

        pltpu.async_copy(peg.at[pl.ds(base, CE), pl.ds(c64, 64)], sig[p], sem_in[p])
        pltpu.async_copy(edge.at[pl.ds(base, CE), pl.ds(c64, 64)], yo[p], sem_in[p])

    def wait_in(p):
        pltpu.make_async_copy(ti0.at[ib[0]], g_sg[p], sem_in[p]).wait()
        pltpu.make_async_copy(tj0.at[jb[0]], g_dj[p], sem_in[p]).wait()
        pltpu.make_async_copy(peg.at[pl.ds(0, CE), pl.ds(c64, 64)], sig[p], sem_in[p]).wait()
        pltpu.make_async_copy(edge.at[pl.ds(0, CE), pl.ds(c64, 64)], yo[p], sem_in[p]).wait()

    def compute(p):
        def row2(t, x_):
            vals = []
            for rr in range(2):
                r = 2 * t + rr
                ys = [g_sg[p][r, pl.ds(q * 16, 16)]
                      + g_dj[p][r, pl.ds(q * 16, 16)]
                      + sig[p][r, pl.ds(q * 16, 16)] for q in range(4)]
                es = [yo[p][r, pl.ds(q * 16, 16)] for q in range(4)]
                dus = [g_dj[p][r, pl.ds(64 + q * 16, 16)] for q in range(4)]
                vals.append((r, ys, es, dus))
            for r, ys, es, dus in vals:
                for q in range(4):
                    sl = pl.ds(q * 16, 16)
                    sg = 1.0 / (1.0 + jnp.exp(-ys[q]))
                    sig[p][r, sl] = sg
                    mb[p][r, sl] = dus[q] * sg
                    yo[p][r, sl] = es[q] + ys[q] * sg
            return x_
        lax.fori_loop(0, CE // 2, row2, None)

    def issue_out(k, p, q4):
        base = ebase0 + k * CE
        pltpu.async_copy(yo[p], yout.at[pl.ds(base, CE), pl.ds(c64, 64)], sem_out[p])
        pltpu.sync_copy(sig[p], acc_sig.at[ib[q4]], add=True)
        pltpu.sync_copy(mb[p], acc_m.at[ib[q4]], add=True)

    def wait_out(p):
        pltpu.make_async_copy(yo[p], yout.at[pl.ds(0, CE), pl.ds(c64, 64)], sem_out[p]).wait()

    issue_ij(0, 0)
    issue_ij(1, 1)
    wait_ij(0)
    issue_in(0, 0, 0)

    wait_in(0)
    compute(0)
    issue_out(0, 0, 0)
    wait_ij(1)
    issue_in(1, 1, 1)
    issue_ij(2, 2)

    wait_in(1)
    compute(1)
    issue_out(1, 1, 1)
    wait_ij(2)
    wait_out(0)
    issue_in(2, 0, 2)
    issue_ij(3, 3)

    def superstep(t, x_):
        k0 = 2 + t * 4

        def stage(koff, p, q4):
            k = k0 + koff
            wait_in(p)
            compute(p)
            issue_out(k, p, q4)
            wait_ij((q4 + 1) % 4)
            wait_out(1 - p)
            issue_in(k + 1, 1 - p, (q4 + 1) % 4)
            issue_ij(k + 2, (q4 + 2) % 4)

        stage(0, 0, 2)
        stage(1, 1, 3)
        stage(2, 0, 0)
        stage(3, 1, 1)
        return x_

    lax.fori_loop(0, (nchunks - 4) // 4, superstep, None)

    wait_in(0)
    compute(0)
    issue_out(nchunks - 2, 0, 2)
    wait_ij(3)
    wait_out(1)
    issue_in(nchunks - 1, 1, 3)

    wait_in(1)
    compute(1)
    issue_out(nchunks - 1, 1, 3)

    wait_out(0)
    wait_out(1)
    plsc.subcore_barrier()

    nbase0 = s * npt

    def ep(q, x_):
        nb = nbase0 + q * NODE_SUB
        pltpu.sync_copy(acc_sig.at[pl.ds(nb, NODE_SUB)], ep_sig)
        pltpu.sync_copy(acc_m.at[pl.ds(nb, NODE_SUB)], ep_m)
        pltpu.sync_copy(psut.at[pl.ds(cN + nb, NODE_SUB)], ep_psu)
        pltpu.sync_copy(node_t.at[pl.ds(cN + nb, NODE_SUB)], ep_node)

        def nrow(r, y_):
            sig4 = [ep_sig[r, pl.ds(q4 * 16, 16)] for q4 in range(4)]
            m4 = [ep_m[r, pl.ds(q4 * 16, 16)] for q4 in range(4)]
            psu4 = [ep_psu[r, pl.ds(q4 * 16, 16)] for q4 in range(4)]
            nd4 = [ep_node[r, pl.ds(q4 * 16, 16)] for q4 in range(4)]
            for q4 in range(4):
                t = psu4[q4] + m4[q4] / (sig4[q4] + 1e-6)
                st = 1.0 / (1.0 + jnp.exp(-t))
                ep_node[r, pl.ds(q4 * 16, 16)] = nd4[q4] + t * st
            return y_
        lax.fori_loop(0, NODE_SUB, nrow, None)

        pltpu.sync_copy(ep_node, xout.at[pl.ds(cN + nb, NODE_SUB)])
        return x_

    lax.fori_loop(0, nsub, ep, None)


def kernel(node_feats, edge_feats, i, j, W_sg, b_sg, W_dg, b_dg,
           W_eg, b_eg, W_su, b_su, W_du, b_du):
    B, N, D = node_feats.shape
    E = edge_feats.shape[1]
    node2 = node_feats.reshape(N, D)
    edge2 = edge_feats.reshape(E, D)
    i32 = i.astype(jnp.int32)
    j32 = j.astype(jnp.int32)
    h = D // 2

    BN = 400
    wspec = pl.BlockSpec((D, D), lambda nb: (0, 0))
    bspec = pl.BlockSpec((1, D), lambda nb: (0, 0))
    outs = pl.pallas_call(
        _node_proj_body,
        grid=(N // BN,),
        in_specs=[pl.BlockSpec((BN, D), lambda nb: (nb, 0)),
                  wspec, bspec, wspec, bspec, wspec, bspec, wspec, bspec],
        out_specs=[pl.BlockSpec((BN, D), lambda nb: (nb, 0))] * 4,
        out_shape=[jax.ShapeDtypeStruct((N, D), F32)] * 4,
    )(node2, W_sg, b_sg.reshape(1, D), W_dg, b_dg.reshape(1, D),
      W_du, b_du.reshape(1, D), W_su, b_su.reshape(1, D))
    p_sg, p_dg, p_du, p_su = outs

    ti0 = p_sg[:, :h]
    ti1 = p_sg[:, h:]
    tj0 = jnp.concatenate([p_dg[:, :h], p_du[:, :h]], axis=1)
    tj1 = jnp.concatenate([p_dg[:, h:], p_du[:, h:]], axis=1)
    psut = p_su.reshape(N, 2, h).transpose(1, 0, 2).reshape(2 * N, h)
    node_t = node2.reshape(N, 2, h).transpose(1, 0, 2).reshape(2 * N, h)

    BE = 1000
    peg = pl.pallas_call(
        _edge_proj_body,
        grid=(E // BE,),
        in_specs=[pl.BlockSpec((BE, D), lambda nb: (nb, 0)),
                  pl.BlockSpec((D, D), lambda nb: (0, 0)),
                  pl.BlockSpec((1, D), lambda nb: (0, 0))],
        out_specs=pl.BlockSpec((BE, D), lambda nb: (nb, 0)),
        out_shape=jax.ShapeDtypeStruct((E, D), F32),
    )(edge2, W_eg, b_eg.reshape(1, D))

    mesh = plsc.VectorSubcoreMesh(core_axis_name="c", subcore_axis_name="s",
                                  num_cores=NC, num_subcores=NS)
    sc_fn = pl.kernel(
        functools.partial(_sc_body, N, E),
        out_type=[jax.ShapeDtypeStruct((E, D), F32),
                  jax.ShapeDtypeStruct((2 * N, h), F32)],
        mesh=mesh,
        compiler_params=pltpu.CompilerParams(use_tc_tiling_on_sc=False),
        scratch_types=[
            pltpu.VMEM_SHARED((N, h), F32),
            pltpu.VMEM_SHARED((N, h), F32),
            pltpu.VMEM((CE,), jnp.int32),
            pltpu.VMEM((CE,), jnp.int32),
            pltpu.VMEM((CE,), jnp.int32),
            pltpu.VMEM((CE,), jnp.int32),
            pltpu.VMEM((CE,), jnp.int32),
            pltpu.VMEM((CE,), jnp.int32),
            pltpu.VMEM((CE,), jnp.int32),
            pltpu.VMEM((CE,), jnp.int32),
            pltpu.VMEM((CE, h), F32),
            pltpu.VMEM((CE, h), F32),
            pltpu.VMEM((CE, D), F32),
            pltpu.VMEM((CE, D), F32),
            pltpu.VMEM((CE, h), F32),
            pltpu.VMEM((CE, h), F32),
            pltpu.VMEM((CE, h), F32),
            pltpu.VMEM((CE, h), F32),
            pltpu.VMEM((CE, h), F32),
            pltpu.VMEM((CE, h), F32),
            pltpu.VMEM((NODE_SUB, h), F32),
            pltpu.VMEM((NODE_SUB, h), F32),
            pltpu.VMEM((NODE_SUB, h), F32),
            pltpu.VMEM((NODE_SUB, h), F32),
            pltpu.SemaphoreType.DMA,
            pltpu.SemaphoreType.DMA,
            pltpu.SemaphoreType.DMA,
            pltpu.SemaphoreType.DMA,
            pltpu.SemaphoreType.DMA,
        ],
    )
    yout, xout = sc_fn(ti0, ti1, tj0, tj1, psut, node_t, peg, edge2, i32, j32)

    x = xout.reshape(2, N, h).transpose(1, 0, 2).reshape(B, N, D)
    y = yout.reshape(B, E, D)
    return (x, y)


# speedup vs baseline: 2.6581x; 1.0054x over previous
import functools

import jax
import jax.numpy as jnp
from jax import lax
from jax.experimental import pallas as pl
from jax.experimental.pallas import tpu as pltpu
from jax.experimental.pallas import tpu_sc as plsc

F32 = jnp.float32
NC, NS = 2, 16
CE = 40
NODE_SUB = 25


def _node_proj_body(x, wsg, bsg, wdg, bdg, wdu, bdu, wsu, bsu,
                    osg, odg, odu, osu):
    xv = x[...]

    def proj(w, b):
        return lax.dot_general(xv, w[...], (((1,), (1,)), ((), ())),
                               preferred_element_type=F32) + b[...]

    osg[...] = proj(wsg, bsg)
    odg[...] = proj(wdg, bdg)
    odu[...] = proj(wdu, bdu)
    osu[...] = proj(wsu, bsu)


def _edge_proj_body(x, w, b, o):
    o[...] = lax.dot_general(x[...], w[...], (((1,), (1,)), ((), ())),
                             preferred_element_type=F32) + b[...]


def _sc_body(N, E, ti0, ti1, tj0, tj1, psut, node_t, peg, edge, i_hbm, j_hbm,
             yout, xout,
             acc2,
             ib0, ib1, ib2, ib3, jb0, jb1, jb2, jb3,
             g_sg0, g_sg1, g_dj0, g_dj1,
             sm0, sm1, pg0, pg1, yo0, yo1,
             ep_sm, ep_psu, ep_node,
             sem_ij, sem_in0, sem_in1, sem_out0, sem_out1):
    c = lax.axis_index("c")
    s = lax.axis_index("s")
    cN = c * N
    c64 = c * 64
    zero16 = jnp.zeros((16,), F32)

    ib = (ib0, ib1, ib2, ib3)
    jb = (jb0, jb1, jb2, jb3)
    g_sg = (g_sg0, g_sg1)
    g_dj = (g_dj0, g_dj1)
    sm = (sm0, sm1)
    pg = (pg0, pg1)
    yo = (yo0, yo1)
    sem_in = (sem_in0, sem_in1)
    sem_out = (sem_out0, sem_out1)

    npt = N // NS
    ept = E // NS
    nchunks = ept // CE
    nsub = npt // NODE_SUB
    ebase0 = s * ept

    def zrow(r, x_):
        for q in range(8):
            ep_sm[r, pl.ds(q * 16, 16)] = zero16
        return x_
    lax.fori_loop(0, NODE_SUB, zrow, None)
    for q in range(nsub):
        pltpu.sync_copy(ep_sm, acc2.at[pl.ds(s * npt + q * NODE_SUB, NODE_SUB)])
    plsc.subcore_barrier()

    def issue_ij(k, q4):
        base = ebase0 + k * CE
        pltpu.async_copy(i_hbm.at[pl.ds(base, CE)], ib[q4], sem_ij)
        pltpu.async_copy(j_hbm.at[pl.ds(base, CE)], jb[q4], sem_ij)

    def wait_ij(q4):
        pltpu.make_async_copy(i_hbm.at[pl.ds(0, CE)], ib[q4], sem_ij).wait()
        pltpu.make_async_copy(j_hbm.at[pl.ds(0, CE)], jb[q4], sem_ij).wait()

    def issue_in(k, p, q4):
        base = ebase0 + k * CE

        @pl.when(c == 0)
        def _():
            pltpu.async_copy(ti0.at[ib[q4]], g_sg[p], sem_in[p])
            pltpu.async_copy(tj0.at[jb[q4]], g_dj[p], sem_in[p])

        @pl.when(c == 1)
        def _():
            pltpu.async_copy(ti1.at[ib[q4]], g_sg[p], sem_in[p])
            pltpu.async_copy(tj1.at[jb[q4]], g_dj[p], sem_in[p])

        pltpu.async_copy(peg.at[pl.ds(base, CE), pl.ds(c64, 64)], pg[p], sem_in[p])
        pltpu.async_copy(edge.at[pl.ds(base, CE), pl.ds(c64, 64)], yo[p], sem_in[p])

    def wait_in(p):
        pltpu.make_async_copy(ti0.at[ib[0]], g_sg[p], sem_in[p]).wait()
        pltpu.make_async_copy(tj0.at[jb[0]], g_dj[p], sem_in[p]).wait()
        pltpu.make_async_copy(peg.at[pl.ds(0, CE), pl.ds(c64, 64)], pg[p], sem_in[p]).wait()
        pltpu.make_async_copy(edge.at[pl.ds(0, CE), pl.ds(c64, 64)], yo[p], sem_in[p]).wait()

    def compute(p):
        def row2(t, x_):
            vals = []
            for rr in range(2):
                r = 2 * t + rr
                ys = [g_sg[p][r, pl.ds(q * 16, 16)]
                      + g_dj[p][r, pl.ds(q * 16, 16)]
                      + pg[p][r, pl.ds(q * 16, 16)] for q in range(4)]
                es = [yo[p][r, pl.ds(q * 16, 16)] for q in range(4)]
                dus = [g_dj[p][r, pl.ds(64 + q * 16, 16)] for q in range(4)]
                vals.append((r, ys, es, dus))
            for r, ys, es, dus in vals:
                for q in range(4):
                    sg = 1.0 / (1.0 + jnp.exp(-ys[q]))
                    sm[p][r, pl.ds(q * 16, 16)] = sg
                    sm[p][r, pl.ds(64 + q * 16, 16)] = dus[q] * sg
                    yo[p][r, pl.ds(q * 16, 16)] = es[q] + ys[q] * sg
            return x_
        lax.fori_loop(0, CE // 2, row2, None)

    def issue_out(k, p, q4):
        base = ebase0 + k * CE
        pltpu.async_copy(yo[p], yout.at[pl.ds(base, CE), pl.ds(c64, 64)], sem_out[p])
        pltpu.sync_copy(sm[p], acc2.at[ib[q4]], add=True)

    def wait_out(p):
        pltpu.make_async_copy(yo[p], yout.at[pl.ds(0, CE), pl.ds(c64, 64)], sem_out[p]).wait()

    issue_ij(0, 0)
    issue_ij(1, 1)
    wait_ij(0)
    issue_in(0, 0, 0)

    wait_in(0)
    compute(0)
    issue_out(0, 0, 0)
    wait_ij(1)
    issue_in(1, 1, 1)
    issue_ij(2, 2)

    wait_in(1)
    compute(1)
    issue_out(1, 1, 1)
    wait_ij(2)
    wait_out(0)
    issue_in(2, 0, 2)
    issue_ij(3, 3)

    def superstep(t, x_):
        k0 = 2 + t * 4

        def stage(koff, p, q4):
            k = k0 + koff
            wait_in(p)
            compute(p)
            issue_out(k, p, q4)
            wait_ij((q4 + 1) % 4)
            wait_out(1 - p)
            issue_in(k + 1, 1 - p, (q4 + 1) % 4)
            issue_ij(k + 2, (q4 + 2) % 4)

        stage(0, 0, 2)
        stage(1, 1, 3)
        stage(2, 0, 0)
        stage(3, 1, 1)
        return x_

    lax.fori_loop(0, (nchunks - 4) // 4, superstep, None)

    wait_in(0)
    compute(0)
    issue_out(nchunks - 2, 0, 2)
    wait_ij(3)
    wait_out(1)
    issue_in(nchunks - 1, 1, 3)

    wait_in(1)
    compute(1)
    issue_out(nchunks - 1, 1, 3)

    wait_out(0)
    wait_out(1)
    plsc.subcore_barrier()

    nbase0 = s * npt

    def ep(q, x_):
        nb = nbase0 + q * NODE_SUB
        pltpu.sync_copy(acc2.at[pl.ds(nb, NODE_SUB)], ep_sm)
        pltpu.sync_copy(psut.at[pl.ds(cN + nb, NODE_SUB)], ep_psu)
        pltpu.sync_copy(node_t.at[pl.ds(cN + nb, NODE_SUB)], ep_node)

        def nrow(r, y_):
            sig4 = [ep_sm[r, pl.ds(q4 * 16, 16)] for q4 in range(4)]
            m4 = [ep_sm[r, pl.ds(64 + q4 * 16, 16)] for q4 in range(4)]
            psu4 = [ep_psu[r, pl.ds(q4 * 16, 16)] for q4 in range(4)]
            nd4 = [ep_node[r, pl.ds(q4 * 16, 16)] for q4 in range(4)]
            for q4 in range(4):
                t = psu4[q4] + m4[q4] / (sig4[q4] + 1e-6)
                st = 1.0 / (1.0 + jnp.exp(-t))
                ep_node[r, pl.ds(q4 * 16, 16)] = nd4[q4] + t * st
            return y_
        lax.fori_loop(0, NODE_SUB, nrow, None)

        pltpu.sync_copy(ep_node, xout.at[pl.ds(cN + nb, NODE_SUB)])
        return x_

    lax.fori_loop(0, nsub, ep, None)


def kernel(node_feats, edge_feats, i, j, W_sg, b_sg, W_dg, b_dg,
           W_eg, b_eg, W_su, b_su, W_du, b_du):
    B, N, D = node_feats.shape
    E = edge_feats.shape[1]
    node2 = node_feats.reshape(N, D)
    edge2 = edge_feats.reshape(E, D)
    i32 = i.astype(jnp.int32)
    j32 = j.astype(jnp.int32)
    h = D // 2

    BN = 400
    wspec = pl.BlockSpec((D, D), lambda nb: (0, 0))
    bspec = pl.BlockSpec((1, D), lambda nb: (0, 0))
    outs = pl.pallas_call(
        _node_proj_body,
        grid=(N // BN,),
        in_specs=[pl.BlockSpec((BN, D), lambda nb: (nb, 0)),
                  wspec, bspec, wspec, bspec, wspec, bspec, wspec, bspec],
        out_specs=[pl.BlockSpec((BN, D), lambda nb: (nb, 0))] * 4,
        out_shape=[jax.ShapeDtypeStruct((N, D), F32)] * 4,
    )(node2, W_sg, b_sg.reshape(1, D), W_dg, b_dg.reshape(1, D),
      W_du, b_du.reshape(1, D), W_su, b_su.reshape(1, D))
    p_sg, p_dg, p_du, p_su = outs

    ti0 = p_sg[:, :h]
    ti1 = p_sg[:, h:]
    tj0 = jnp.concatenate([p_dg[:, :h], p_du[:, :h]], axis=1)
    tj1 = jnp.concatenate([p_dg[:, h:], p_du[:, h:]], axis=1)
    psut = p_su.reshape(N, 2, h).transpose(1, 0, 2).reshape(2 * N, h)
    node_t = node2.reshape(N, 2, h).transpose(1, 0, 2).reshape(2 * N, h)

    BE = 1000
    peg = pl.pallas_call(
        _edge_proj_body,
        grid=(E // BE,),
        in_specs=[pl.BlockSpec((BE, D), lambda nb: (nb, 0)),
                  pl.BlockSpec((D, D), lambda nb: (0, 0)),
                  pl.BlockSpec((1, D), lambda nb: (0, 0))],
        out_specs=pl.BlockSpec((BE, D), lambda nb: (nb, 0)),
        out_shape=jax.ShapeDtypeStruct((E, D), F32),
    )(edge2, W_eg, b_eg.reshape(1, D))

    mesh = plsc.VectorSubcoreMesh(core_axis_name="c", subcore_axis_name="s",
                                  num_cores=NC, num_subcores=NS)
    sc_fn = pl.kernel(
        functools.partial(_sc_body, N, E),
        out_type=[jax.ShapeDtypeStruct((E, D), F32),
                  jax.ShapeDtypeStruct((2 * N, h), F32)],
        mesh=mesh,
        compiler_params=pltpu.CompilerParams(use_tc_tiling_on_sc=False),
        scratch_types=[
            pltpu.VMEM_SHARED((N, D), F32),
            pltpu.VMEM((CE,), jnp.int32),
            pltpu.VMEM((CE,), jnp.int32),
            pltpu.VMEM((CE,), jnp.int32),
            pltpu.VMEM((CE,), jnp.int32),
            pltpu.VMEM((CE,), jnp.int32),
            pltpu.VMEM((CE,), jnp.int32),
            pltpu.VMEM((CE,), jnp.int32),
            pltpu.VMEM((CE,), jnp.int32),
            pltpu.VMEM((CE, h), F32),
            pltpu.VMEM((CE, h), F32),
            pltpu.VMEM((CE, D), F32),
            pltpu.VMEM((CE, D), F32),
            pltpu.VMEM((CE, D), F32),
            pltpu.VMEM((CE, D), F32),
            pltpu.VMEM((CE, h), F32),
            pltpu.VMEM((CE, h), F32),
            pltpu.VMEM((CE, h), F32),
            pltpu.VMEM((CE, h), F32),
            pltpu.VMEM((NODE_SUB, D), F32),
            pltpu.VMEM((NODE_SUB, h), F32),
            pltpu.VMEM((NODE_SUB, h), F32),
            pltpu.SemaphoreType.DMA,
            pltpu.SemaphoreType.DMA,
            pltpu.SemaphoreType.DMA,
            pltpu.SemaphoreType.DMA,
            pltpu.SemaphoreType.DMA,
        ],
    )
    yout, xout = sc_fn(ti0, ti1, tj0, tj1, psut, node_t, peg, edge2, i32, j32)

    x = xout.reshape(2, N, h).transpose(1, 0, 2).reshape(B, N, D)
    y = yout.reshape(B, E, D)
    return (x, y)


# speedup vs baseline: 3.4308x; 1.2907x over previous
import functools

import jax
import jax.numpy as jnp
from jax import lax
from jax.experimental import pallas as pl
from jax.experimental.pallas import tpu as pltpu
from jax.experimental.pallas import tpu_sc as plsc

F32 = jnp.float32
NC, NS = 2, 16
CE = 40
NODE_SUB = 25


def _node_proj_body(x, wsg, bsg, wdg, bdg, wdu, bdu, wsu, bsu,
                    osg, odg, odu, osu):
    xv = x[...]

    def proj(w, b):
        return lax.dot_general(xv, w[...], (((1,), (1,)), ((), ())),
                               preferred_element_type=F32) + b[...]

    osg[...] = proj(wsg, bsg)
    odg[...] = proj(wdg, bdg)
    odu[...] = proj(wdu, bdu)
    osu[...] = proj(wsu, bsu)


def _edge_proj_body(x, w, b, o):
    o[...] = lax.dot_general(x[...], w[...], (((1,), (1,)), ((), ())),
                             preferred_element_type=F32) + b[...]


def _sc_body(N, E, ti0, ti1, tj0, tj1, psut, node_t, peg, edge, i_hbm, j_hbm,
             yout, xout,
             acc2,
             ib0, ib1, ib2, ib3, jb0, jb1, jb2, jb3,
             g_sg0, g_sg1, g_dj0, g_dj1,
             sm0, sm1, pg0, pg1, yo0, yo1,
             ep_sm, ep_psu, ep_node,
             sem_ij, sem_in0, sem_in1, sem_out0, sem_out1):
    c = lax.axis_index("c")
    s = lax.axis_index("s")
    cN = c * N
    c64 = c * 64
    zero16 = jnp.zeros((16,), F32)

    ib = (ib0, ib1, ib2, ib3)
    jb = (jb0, jb1, jb2, jb3)
    g_sg = (g_sg0, g_sg1)
    g_dj = (g_dj0, g_dj1)
    sm = (sm0, sm1)
    pg = (pg0, pg1)
    yo = (yo0, yo1)
    sem_in = (sem_in0, sem_in1)
    sem_out = (sem_out0, sem_out1)

    npt = N // NS
    ept = E // NS
    nchunks = ept // CE
    nsub = npt // NODE_SUB
    ebase0 = s * ept

    def zrow(r, x_):
        for q in range(8):
            ep_sm[r, pl.ds(q * 16, 16)] = zero16
        return x_
    lax.fori_loop(0, NODE_SUB, zrow, None)
    for q in range(nsub):
        pltpu.sync_copy(ep_sm, acc2.at[pl.ds(s * npt + q * NODE_SUB, NODE_SUB)])
    plsc.subcore_barrier()

    def issue_ij(k, q4):
        base = ebase0 + k * CE
        pltpu.async_copy(i_hbm.at[pl.ds(base, CE)], ib[q4], sem_ij)
        pltpu.async_copy(j_hbm.at[pl.ds(base, CE)], jb[q4], sem_ij)

    def wait_ij(q4):
        pltpu.make_async_copy(i_hbm.at[pl.ds(0, CE)], ib[q4], sem_ij).wait()
        pltpu.make_async_copy(j_hbm.at[pl.ds(0, CE)], jb[q4], sem_ij).wait()

    def issue_in(k, p, q4):
        base = ebase0 + k * CE

        @pl.when(c == 0)
        def _():
            pltpu.async_copy(ti0.at[ib[q4]], g_sg[p], sem_in[p])
            pltpu.async_copy(tj0.at[jb[q4]], g_dj[p], sem_in[p])

        @pl.when(c == 1)
        def _():
            pltpu.async_copy(ti1.at[ib[q4]], g_sg[p], sem_in[p])
            pltpu.async_copy(tj1.at[jb[q4]], g_dj[p], sem_in[p])

        pltpu.async_copy(peg.at[pl.ds(base, CE), pl.ds(c64, 64)], pg[p], sem_in[p])
        pltpu.async_copy(edge.at[pl.ds(base, CE), pl.ds(c64, 64)], yo[p], sem_in[p])

    def wait_in(p):
        pltpu.make_async_copy(ti0.at[ib[0]], g_sg[p], sem_in[p]).wait()
        pltpu.make_async_copy(tj0.at[jb[0]], g_dj[p], sem_in[p]).wait()
        pltpu.make_async_copy(peg.at[pl.ds(0, CE), pl.ds(c64, 64)], pg[p], sem_in[p]).wait()
        pltpu.make_async_copy(edge.at[pl.ds(0, CE), pl.ds(c64, 64)], yo[p], sem_in[p]).wait()

    def compute(p):
        def row2(t, x_):
            vals = []
            for rr in range(2):
                r = 2 * t + rr
                ys = [g_sg[p][r, pl.ds(q * 16, 16)]
                      + g_dj[p][r, pl.ds(q * 16, 16)]
                      + pg[p][r, pl.ds(q * 16, 16)] for q in range(4)]
                es = [yo[p][r, pl.ds(q * 16, 16)] for q in range(4)]
                dus = [g_dj[p][r, pl.ds(64 + q * 16, 16)] for q in range(4)]
                vals.append((r, ys, es, dus))
            for r, ys, es, dus in vals:
                for q in range(4):
                    sg = 1.0 / (1.0 + jnp.exp(-ys[q]))
                    sm[p][r, pl.ds(q * 16, 16)] = sg
                    sm[p][r, pl.ds(64 + q * 16, 16)] = dus[q] * sg
                    yo[p][r, pl.ds(q * 16, 16)] = es[q] + ys[q] * sg
            return x_
        lax.fori_loop(0, CE // 2, row2, None)

    def issue_out(k, p, q4):
        base = ebase0 + k * CE
        pltpu.async_copy(yo[p], yout.at[pl.ds(base, CE), pl.ds(c64, 64)], sem_out[p])
        pltpu.sync_copy(sm[p], acc2.at[ib[q4]], add=True)

    def wait_out(p):
        pltpu.make_async_copy(yo[p], yout.at[pl.ds(0, CE), pl.ds(c64, 64)], sem_out[p]).wait()

    issue_ij(0, 0)
    issue_ij(1, 1)
    wait_ij(0)
    issue_in(0, 0, 0)

    wait_ij(1)
    issue_in(1, 1, 1)
    wait_in(0)
    compute(0)
    issue_out(0, 0, 0)
    issue_ij(2, 2)

    wait_ij(2)
    wait_out(0)
    issue_in(2, 0, 2)
    wait_in(1)
    compute(1)
    issue_out(1, 1, 1)
    issue_ij(3, 3)

    def superstep(t, x_):
        k0 = 2 + t * 4

        def stage(koff, p, q4):
            k = k0 + koff
            wait_ij((q4 + 1) % 4)
            wait_out(1 - p)
            issue_in(k + 1, 1 - p, (q4 + 1) % 4)
            wait_in(p)
            compute(p)
            issue_out(k, p, q4)
            issue_ij(k + 2, (q4 + 2) % 4)

        stage(0, 0, 2)
        stage(1, 1, 3)
        stage(2, 0, 0)
        stage(3, 1, 1)
        return x_

    lax.fori_loop(0, (nchunks - 4) // 4, superstep, None)

    wait_ij(3)
    wait_out(1)
    issue_in(nchunks - 1, 1, 3)
    wait_in(0)
    compute(0)
    issue_out(nchunks - 2, 0, 2)

    wait_in(1)
    compute(1)
    issue_out(nchunks - 1, 1, 3)

    wait_out(0)
    wait_out(1)
    plsc.subcore_barrier()

    nbase0 = s * npt

    def ep(q, x_):
        nb = nbase0 + q * NODE_SUB
        pltpu.sync_copy(acc2.at[pl.ds(nb, NODE_SUB)], ep_sm)
        pltpu.sync_copy(psut.at[pl.ds(cN + nb, NODE_SUB)], ep_psu)
        pltpu.sync_copy(node_t.at[pl.ds(cN + nb, NODE_SUB)], ep_node)

        def nrow(r, y_):
            sig4 = [ep_sm[r, pl.ds(q4 * 16, 16)] for q4 in range(4)]
            m4 = [ep_sm[r, pl.ds(64 + q4 * 16, 16)] for q4 in range(4)]
            psu4 = [ep_psu[r, pl.ds(q4 * 16, 16)] for q4 in range(4)]
            nd4 = [ep_node[r, pl.ds(q4 * 16, 16)] for q4 in range(4)]
            for q4 in range(4):
                t = psu4[q4] + m4[q4] / (sig4[q4] + 1e-6)
                st = 1.0 / (1.0 + jnp.exp(-t))
                ep_node[r, pl.ds(q4 * 16, 16)] = nd4[q4] + t * st
            return y_
        lax.fori_loop(0, NODE_SUB, nrow, None)

        pltpu.sync_copy(ep_node, xout.at[pl.ds(cN + nb, NODE_SUB)])
        return x_

    lax.fori_loop(0, nsub, ep, None)


def kernel(node_feats, edge_feats, i, j, W_sg, b_sg, W_dg, b_dg,
           W_eg, b_eg, W_su, b_su, W_du, b_du):
    B, N, D = node_feats.shape
    E = edge_feats.shape[1]
    node2 = node_feats.reshape(N, D)
    edge2 = edge_feats.reshape(E, D)
    i32 = i.astype(jnp.int32)
    j32 = j.astype(jnp.int32)
    h = D // 2

    BN = 400
    wspec = pl.BlockSpec((D, D), lambda nb: (0, 0))
    bspec = pl.BlockSpec((1, D), lambda nb: (0, 0))
    outs = pl.pallas_call(
        _node_proj_body,
        grid=(N // BN,),
        in_specs=[pl.BlockSpec((BN, D), lambda nb: (nb, 0)),
                  wspec, bspec, wspec, bspec, wspec, bspec, wspec, bspec],
        out_specs=[pl.BlockSpec((BN, D), lambda nb: (nb, 0))] * 4,
        out_shape=[jax.ShapeDtypeStruct((N, D), F32)] * 4,
    )(node2, W_sg, b_sg.reshape(1, D), W_dg, b_dg.reshape(1, D),
      W_du, b_du.reshape(1, D), W_su, b_su.reshape(1, D))
    p_sg, p_dg, p_du, p_su = outs

    ti0 = p_sg[:, :h]
    ti1 = p_sg[:, h:]
    tj0 = jnp.concatenate([p_dg[:, :h], p_du[:, :h]], axis=1)
    tj1 = jnp.concatenate([p_dg[:, h:], p_du[:, h:]], axis=1)
    psut = p_su.reshape(N, 2, h).transpose(1, 0, 2).reshape(2 * N, h)
    node_t = node2.reshape(N, 2, h).transpose(1, 0, 2).reshape(2 * N, h)

    BE = 1000
    peg = pl.pallas_call(
        _edge_proj_body,
        grid=(E // BE,),
        in_specs=[pl.BlockSpec((BE, D), lambda nb: (nb, 0)),
                  pl.BlockSpec((D, D), lambda nb: (0, 0)),
                  pl.BlockSpec((1, D), lambda nb: (0, 0))],
        out_specs=pl.BlockSpec((BE, D), lambda nb: (nb, 0)),
        out_shape=jax.ShapeDtypeStruct((E, D), F32),
    )(edge2, W_eg, b_eg.reshape(1, D))

    mesh = plsc.VectorSubcoreMesh(core_axis_name="c", subcore_axis_name="s",
                                  num_cores=NC, num_subcores=NS)
    sc_fn = pl.kernel(
        functools.partial(_sc_body, N, E),
        out_type=[jax.ShapeDtypeStruct((E, D), F32),
                  jax.ShapeDtypeStruct((2 * N, h), F32)],
        mesh=mesh,
        compiler_params=pltpu.CompilerParams(use_tc_tiling_on_sc=False),
        scratch_types=[
            pltpu.VMEM_SHARED((N, D), F32),
            pltpu.VMEM((CE,), jnp.int32),
            pltpu.VMEM((CE,), jnp.int32),
            pltpu.VMEM((CE,), jnp.int32),
            pltpu.VMEM((CE,), jnp.int32),
            pltpu.VMEM((CE,), jnp.int32),
            pltpu.VMEM((CE,), jnp.int32),
            pltpu.VMEM((CE,), jnp.int32),
            pltpu.VMEM((CE,), jnp.int32),
            pltpu.VMEM((CE, h), F32),
            pltpu.VMEM((CE, h), F32),
            pltpu.VMEM((CE, D), F32),
            pltpu.VMEM((CE, D), F32),
            pltpu.VMEM((CE, D), F32),
            pltpu.VMEM((CE, D), F32),
            pltpu.VMEM((CE, h), F32),
            pltpu.VMEM((CE, h), F32),
            pltpu.VMEM((CE, h), F32),
            pltpu.VMEM((CE, h), F32),
            pltpu.VMEM((NODE_SUB, D), F32),
            pltpu.VMEM((NODE_SUB, h), F32),
            pltpu.VMEM((NODE_SUB, h), F32),
            pltpu.SemaphoreType.DMA,
            pltpu.SemaphoreType.DMA,
            pltpu.SemaphoreType.DMA,
            pltpu.SemaphoreType.DMA,
            pltpu.SemaphoreType.DMA,
        ],
    )
    yout, xout = sc_fn(ti0, ti1, tj0, tj1, psut, node_t, peg, edge2, i32, j32)

    x = xout.reshape(2, N, h).transpose(1, 0, 2).reshape(B, N, D)
    y = yout.reshape(B, E, D)
    return (x, y)


# speedup vs baseline: 3.7758x; 1.1006x over previous
import functools

import jax
import jax.numpy as jnp
from jax import lax
from jax.experimental import pallas as pl
from jax.experimental.pallas import tpu as pltpu
from jax.experimental.pallas import tpu_sc as plsc

F32 = jnp.float32
NC, NS = 2, 16
CE = 40
NODE_SUB = 25


def _node_proj_body(x, wsg, bsg, wdg, bdg, wdu, bdu, wsu, bsu,
                    osg, odg, odu, osu):
    xv = x[...]

    def proj(w, b):
        return lax.dot_general(xv, w[...], (((1,), (1,)), ((), ())),
                               preferred_element_type=F32) + b[...]

    osg[...] = proj(wsg, bsg)
    odg[...] = proj(wdg, bdg)
    odu[...] = proj(wdu, bdu)
    osu[...] = proj(wsu, bsu)


def _edge_proj_body(x, w, b, o):
    o[...] = lax.dot_general(x[...], w[...], (((1,), (1,)), ((), ())),
                             preferred_element_type=F32) + b[...]


def _sc_body(N, E, ti0, ti1, tj0, tj1, psut, node_t, peg, edge, i_hbm, j_hbm,
             yout, xout,
             acc2,
             ib0, ib1, ib2, ib3, jb0, jb1, jb2, jb3,
             g_sg0, g_sg1, g_dj0, g_dj1,
             sm0, sm1, pg0, pg1, yo0, yo1,
             ep_sm, ep_psu, ep_node,
             sem_ij, sem_in0, sem_in1, sem_out0, sem_out1, sem_sc0, sem_sc1):
    c = lax.axis_index("c")
    s = lax.axis_index("s")
    cN = c * N
    c64 = c * 64
    zero16 = jnp.zeros((16,), F32)

    ib = (ib0, ib1, ib2, ib3)
    jb = (jb0, jb1, jb2, jb3)
    g_sg = (g_sg0, g_sg1)
    g_dj = (g_dj0, g_dj1)
    sm = (sm0, sm1)
    pg = (pg0, pg1)
    yo = (yo0, yo1)
    sem_in = (sem_in0, sem_in1)
    sem_out = (sem_out0, sem_out1)
    sem_sc = (sem_sc0, sem_sc1)

    npt = N // NS
    ept = E // NS
    nchunks = ept // CE
    nsub = npt // NODE_SUB
    ebase0 = s * ept

    def zrow(r, x_):
        for q in range(8):
            ep_sm[r, pl.ds(q * 16, 16)] = zero16
        return x_
    lax.fori_loop(0, NODE_SUB, zrow, None)
    for q in range(nsub):
        pltpu.sync_copy(ep_sm, acc2.at[pl.ds(s * npt + q * NODE_SUB, NODE_SUB)])
    plsc.subcore_barrier()

    def issue_ij(k, q4):
        base = ebase0 + k * CE
        pltpu.async_copy(i_hbm.at[pl.ds(base, CE)], ib[q4], sem_ij)
        pltpu.async_copy(j_hbm.at[pl.ds(base, CE)], jb[q4], sem_ij)

    def wait_ij(q4):
        pltpu.make_async_copy(i_hbm.at[pl.ds(0, CE)], ib[q4], sem_ij).wait()
        pltpu.make_async_copy(j_hbm.at[pl.ds(0, CE)], jb[q4], sem_ij).wait()

    def issue_in(k, p, q4):
        base = ebase0 + k * CE

        @pl.when(c == 0)
        def _():
            pltpu.async_copy(ti0.at[ib[q4]], g_sg[p], sem_in[p])
            pltpu.async_copy(tj0.at[jb[q4]], g_dj[p], sem_in[p])

        @pl.when(c == 1)
        def _():
            pltpu.async_copy(ti1.at[ib[q4]], g_sg[p], sem_in[p])
            pltpu.async_copy(tj1.at[jb[q4]], g_dj[p], sem_in[p])

        pltpu.async_copy(peg.at[pl.ds(base, CE), pl.ds(c64, 64)], pg[p], sem_in[p])
        pltpu.async_copy(edge.at[pl.ds(base, CE), pl.ds(c64, 64)], yo[p], sem_in[p])

    def wait_in(p):
        pltpu.make_async_copy(ti0.at[ib[0]], g_sg[p], sem_in[p]).wait()
        pltpu.make_async_copy(tj0.at[jb[0]], g_dj[p], sem_in[p]).wait()
        pltpu.make_async_copy(peg.at[pl.ds(0, CE), pl.ds(c64, 64)], pg[p], sem_in[p]).wait()
        pltpu.make_async_copy(edge.at[pl.ds(0, CE), pl.ds(c64, 64)], yo[p], sem_in[p]).wait()

    def compute(p):
        def row2(t, x_):
            vals = []
            for rr in range(2):
                r = 2 * t + rr
                ys = [g_sg[p][r, pl.ds(q * 16, 16)]
                      + g_dj[p][r, pl.ds(q * 16, 16)]
                      + pg[p][r, pl.ds(q * 16, 16)] for q in range(4)]
                es = [yo[p][r, pl.ds(q * 16, 16)] for q in range(4)]
                dus = [g_dj[p][r, pl.ds(64 + q * 16, 16)] for q in range(4)]
                vals.append((r, ys, es, dus))
            for r, ys, es, dus in vals:
                for q in range(4):
                    sg = 1.0 / (1.0 + jnp.exp(-ys[q]))
                    sm[p][r, pl.ds(q * 16, 16)] = sg
                    sm[p][r, pl.ds(64 + q * 16, 16)] = dus[q] * sg
                    yo[p][r, pl.ds(q * 16, 16)] = es[q] + ys[q] * sg
            return x_
        lax.fori_loop(0, CE // 2, row2, None)

    def issue_out(k, p, q4):
        base = ebase0 + k * CE
        pltpu.async_copy(yo[p], yout.at[pl.ds(base, CE), pl.ds(c64, 64)], sem_out[p])
        pltpu.async_copy(sm[p], acc2.at[ib[q4]], sem_sc[p], add=True)

    def wait_out(p):
        pltpu.make_async_copy(yo[p], yout.at[pl.ds(0, CE), pl.ds(c64, 64)], sem_out[p]).wait()

    def wait_sc(p):
        pltpu.make_async_copy(sm[p], acc2.at[ib[0]], sem_sc[p]).wait()

    issue_ij(0, 0)
    issue_ij(1, 1)
    wait_ij(0)
    issue_in(0, 0, 0)

    wait_ij(1)
    issue_in(1, 1, 1)
    wait_in(0)
    compute(0)
    issue_out(0, 0, 0)
    issue_ij(2, 2)

    wait_ij(2)
    wait_out(0)
    issue_in(2, 0, 2)
    wait_in(1)
    compute(1)
    issue_out(1, 1, 1)
    issue_ij(3, 3)

    def superstep(t, x_):
        k0 = 2 + t * 4

        def stage(koff, p, q4):
            k = k0 + koff
            wait_ij((q4 + 1) % 4)
            wait_out(1 - p)
            issue_in(k + 1, 1 - p, (q4 + 1) % 4)
            wait_in(p)
            wait_sc(p)
            compute(p)
            issue_out(k, p, q4)
            issue_ij(k + 2, (q4 + 2) % 4)

        stage(0, 0, 2)
        stage(1, 1, 3)
        stage(2, 0, 0)
        stage(3, 1, 1)
        return x_

    lax.fori_loop(0, (nchunks - 4) // 4, superstep, None)

    wait_ij(3)
    wait_out(1)
    issue_in(nchunks - 1, 1, 3)
    wait_in(0)
    wait_sc(0)
    compute(0)
    issue_out(nchunks - 2, 0, 2)

    wait_in(1)
    wait_sc(1)
    compute(1)
    issue_out(nchunks - 1, 1, 3)

    wait_out(0)
    wait_out(1)
    wait_sc(0)
    wait_sc(1)
    plsc.subcore_barrier()

    nbase0 = s * npt

    def ep(q, x_):
        nb = nbase0 + q * NODE_SUB
        pltpu.sync_copy(acc2.at[pl.ds(nb, NODE_SUB)], ep_sm)
        pltpu.sync_copy(psut.at[pl.ds(cN + nb, NODE_SUB)], ep_psu)
        pltpu.sync_copy(node_t.at[pl.ds(cN + nb, NODE_SUB)], ep_node)

        def nrow(r, y_):
            sig4 = [ep_sm[r, pl.ds(q4 * 16, 16)] for q4 in range(4)]
            m4 = [ep_sm[r, pl.ds(64 + q4 * 16, 16)] for q4 in range(4)]
            psu4 = [ep_psu[r, pl.ds(q4 * 16, 16)] for q4 in range(4)]
            nd4 = [ep_node[r, pl.ds(q4 * 16, 16)] for q4 in range(4)]
            for q4 in range(4):
                t = psu4[q4] + m4[q4] / (sig4[q4] + 1e-6)
                st = 1.0 / (1.0 + jnp.exp(-t))
                ep_node[r, pl.ds(q4 * 16, 16)] = nd4[q4] + t * st
            return y_
        lax.fori_loop(0, NODE_SUB, nrow, None)

        pltpu.sync_copy(ep_node, xout.at[pl.ds(cN + nb, NODE_SUB)])
        return x_

    lax.fori_loop(0, nsub, ep, None)


def kernel(node_feats, edge_feats, i, j, W_sg, b_sg, W_dg, b_dg,
           W_eg, b_eg, W_su, b_su, W_du, b_du):
    B, N, D = node_feats.shape
    E = edge_feats.shape[1]
    node2 = node_feats.reshape(N, D)
    edge2 = edge_feats.reshape(E, D)
    i32 = i.astype(jnp.int32)
    j32 = j.astype(jnp.int32)
    h = D // 2

    BN = 400
    wspec = pl.BlockSpec((D, D), lambda nb: (0, 0))
    bspec = pl.BlockSpec((1, D), lambda nb: (0, 0))
    outs = pl.pallas_call(
        _node_proj_body,
        grid=(N // BN,),
        in_specs=[pl.BlockSpec((BN, D), lambda nb: (nb, 0)),
                  wspec, bspec, wspec, bspec, wspec, bspec, wspec, bspec],
        out_specs=[pl.BlockSpec((BN, D), lambda nb: (nb, 0))] * 4,
        out_shape=[jax.ShapeDtypeStruct((N, D), F32)] * 4,
    )(node2, W_sg, b_sg.reshape(1, D), W_dg, b_dg.reshape(1, D),
      W_du, b_du.reshape(1, D), W_su, b_su.reshape(1, D))
    p_sg, p_dg, p_du, p_su = outs

    ti0 = p_sg[:, :h]
    ti1 = p_sg[:, h:]
    tj0 = jnp.concatenate([p_dg[:, :h], p_du[:, :h]], axis=1)
    tj1 = jnp.concatenate([p_dg[:, h:], p_du[:, h:]], axis=1)
    psut = p_su.reshape(N, 2, h).transpose(1, 0, 2).reshape(2 * N, h)
    node_t = node2.reshape(N, 2, h).transpose(1, 0, 2).reshape(2 * N, h)

    BE = 1000
    peg = pl.pallas_call(
        _edge_proj_body,
        grid=(E // BE,),
        in_specs=[pl.BlockSpec((BE, D), lambda nb: (nb, 0)),
                  pl.BlockSpec((D, D), lambda nb: (0, 0)),
                  pl.BlockSpec((1, D), lambda nb: (0, 0))],
        out_specs=pl.BlockSpec((BE, D), lambda nb: (nb, 0)),
        out_shape=jax.ShapeDtypeStruct((E, D), F32),
    )(edge2, W_eg, b_eg.reshape(1, D))

    mesh = plsc.VectorSubcoreMesh(core_axis_name="c", subcore_axis_name="s",
                                  num_cores=NC, num_subcores=NS)
    sc_fn = pl.kernel(
        functools.partial(_sc_body, N, E),
        out_type=[jax.ShapeDtypeStruct((E, D), F32),
                  jax.ShapeDtypeStruct((2 * N, h), F32)],
        mesh=mesh,
        compiler_params=pltpu.CompilerParams(use_tc_tiling_on_sc=False),
        scratch_types=[
            pltpu.VMEM_SHARED((N, D), F32),
            pltpu.VMEM((CE,), jnp.int32),
            pltpu.VMEM((CE,), jnp.int32),
            pltpu.VMEM((CE,), jnp.int32),
            pltpu.VMEM((CE,), jnp.int32),
            pltpu.VMEM((CE,), jnp.int32),
            pltpu.VMEM((CE,), jnp.int32),
            pltpu.VMEM((CE,), jnp.int32),
            pltpu.VMEM((CE,), jnp.int32),
            pltpu.VMEM((CE, h), F32),
            pltpu.VMEM((CE, h), F32),
            pltpu.VMEM((CE, D), F32),
            pltpu.VMEM((CE, D), F32),
            pltpu.VMEM((CE, D), F32),
            pltpu.VMEM((CE, D), F32),
            pltpu.VMEM((CE, h), F32),
            pltpu.VMEM((CE, h), F32),
            pltpu.VMEM((CE, h), F32),
            pltpu.VMEM((CE, h), F32),
            pltpu.VMEM((NODE_SUB, D), F32),
            pltpu.VMEM((NODE_SUB, h), F32),
            pltpu.VMEM((NODE_SUB, h), F32),
            pltpu.SemaphoreType.DMA,
            pltpu.SemaphoreType.DMA,
            pltpu.SemaphoreType.DMA,
            pltpu.SemaphoreType.DMA,
            pltpu.SemaphoreType.DMA,
            pltpu.SemaphoreType.DMA,
            pltpu.SemaphoreType.DMA,
        ],
    )
    yout, xout = sc_fn(ti0, ti1, tj0, tj1, psut, node_t, peg, edge2, i32, j32)

    x = xout.reshape(2, N, h).transpose(1, 0, 2).reshape(B, N, D)
    y = yout.reshape(B, E, D)
    return (x, y)
